# baseline (speedup 1.0000x reference)
import jax
import jax.numpy as jnp
from jax.experimental import pallas as pl

B = 1024
D_STATE = 1024
D_EMB = 512
A = 1000

BM = 512


def _fused_kernel(state_ref, we_ref, ws_ref, wq_ref, bq_ref,
                  sample_ref, max_ref, arg_ref):
    emb = jnp.dot(state_ref[...], we_ref[...],
                  preferred_element_type=jnp.float32)
    s = (jnp.dot(emb, ws_ref[...], preferred_element_type=jnp.float32)
         + jnp.dot(emb, wq_ref[...], preferred_element_type=jnp.float32)
         + bq_ref[...][None, :])
    sample_ref[...] = s
    max_ref[...] = jnp.max(s, axis=-1)
    arg_ref[...] = jnp.argmax(s, axis=-1).astype(jnp.int32)


def kernel(state, We, Ws, Wq, bq):
    grid = (B // BM,)
    sample, max_val, action = pl.pallas_call(
        _fused_kernel,
        grid=grid,
        in_specs=[
            pl.BlockSpec((BM, D_STATE), lambda i: (i, 0)),
            pl.BlockSpec((D_STATE, D_EMB), lambda i: (0, 0)),
            pl.BlockSpec((D_EMB, A), lambda i: (0, 0)),
            pl.BlockSpec((D_EMB, A), lambda i: (0, 0)),
            pl.BlockSpec((A,), lambda i: (0,)),
        ],
        out_specs=[
            pl.BlockSpec((BM, A), lambda i: (i, 0)),
            pl.BlockSpec((BM,), lambda i: (i,)),
            pl.BlockSpec((BM,), lambda i: (i,)),
        ],
        out_shape=[
            jax.ShapeDtypeStruct((B, A), jnp.float32),
            jax.ShapeDtypeStruct((B,), jnp.float32),
            jax.ShapeDtypeStruct((B,), jnp.int32),
        ],
    )(state, We, Ws, Wq, bq)
    return sample, max_val, action
